# Initial kernel scaffold; baseline (speedup 1.0000x reference)
import jax
import jax.numpy as jnp
from jax.experimental import pallas as pl


def kernel(target, context, negative_word_batch, input_emb, output_emb):
    raise NotImplementedError("write your pallas kernel here")


# speedup vs baseline: 4.8044x; 4.8044x over previous
import functools

import jax
import jax.numpy as jnp
from jax import lax
from jax.experimental import pallas as pl
from jax.experimental.pallas import tpu as pltpu
from jax.experimental.pallas import tpu_sc as plsc

V = 1000000
D = 64
B = 16384
K = 20

NC = 2
NS = 16
NW = NC * NS
BPW = B // NW
CHUNK = 128
NCH = BPW // CHUNK
GRPS = CHUNK // 16


def _dots16(buf, vbuf, scr, g):
    def row(j, _):
        r = g * 16 + j
        p = buf[r, pl.ds(0, 16)] * vbuf[r, pl.ds(0, 16)]
        p = p + buf[r, pl.ds(16, 16)] * vbuf[r, pl.ds(16, 16)]
        p = p + buf[r, pl.ds(32, 16)] * vbuf[r, pl.ds(32, 16)]
        p = p + buf[r, pl.ds(48, 16)] * vbuf[r, pl.ds(48, 16)]
        plsc.store_scatter(scr, [lax.iota(jnp.int32, 16) * 16 + j], p)
        return 0

    lax.fori_loop(0, 16, row, 0)

    def srow(i, a):
        return a + scr[pl.ds(i * 16, 16)]

    return lax.fori_loop(0, 16, srow, jnp.zeros((16,), jnp.float32))


_mesh = plsc.VectorSubcoreMesh(core_axis_name="c", subcore_axis_name="s")


@functools.partial(
    pl.kernel,
    mesh=_mesh,
    compiler_params=pltpu.CompilerParams(
        needs_layout_passes=False, use_tc_tiling_on_sc=False
    ),
    out_type=(
        jax.ShapeDtypeStruct((B,), jnp.float32),
        jax.ShapeDtypeStruct((B,), jnp.float32),
    ),
    scratch_types=[
        pltpu.VMEM((CHUNK,), jnp.int32),
        pltpu.VMEM((CHUNK,), jnp.int32),
        pltpu.VMEM((K, CHUNK), jnp.int32),
        pltpu.VMEM((CHUNK, D), jnp.float32),
        pltpu.VMEM((CHUNK, D), jnp.float32),
        pltpu.VMEM((CHUNK, D), jnp.float32),
        pltpu.VMEM((256,), jnp.float32),
        pltpu.VMEM((CHUNK,), jnp.float32),
        pltpu.VMEM((CHUNK,), jnp.float32),
        pltpu.SemaphoreType.DMA,
    ],
)
def _sc_dots(tgt, ctx, negt, iemb, oemb, pdot, ndot,
             tgti, ctxi, negi, vbuf, ubuf, nbuf, scr, pv, nv, sem):
    wid = lax.axis_index("s") * NC + lax.axis_index("c")
    for c in range(NCH):
        off = wid * BPW + c * CHUNK
        pltpu.sync_copy(tgt.at[pl.ds(off, CHUNK)], tgti)
        pltpu.sync_copy(ctx.at[pl.ds(off, CHUNK)], ctxi)
        pltpu.sync_copy(negt.at[:, pl.ds(off, CHUNK)], negi)
        pltpu.async_copy(iemb.at[tgti], vbuf, sem).wait()
        pltpu.async_copy(oemb.at[ctxi], ubuf, sem).wait()

        def pgrp(g, _):
            pv[pl.ds(g * 16, 16)] = _dots16(ubuf, vbuf, scr, g)
            return 0

        lax.fori_loop(0, GRPS, pgrp, 0)

        def zb(g, _):
            nv[pl.ds(g * 16, 16)] = jnp.zeros((16,), jnp.float32)
            return 0

        lax.fori_loop(0, GRPS, zb, 0)

        def kb(k, _):
            pltpu.async_copy(oemb.at[negi.at[k]], nbuf, sem).wait()

            def ngrp(g, _):
                nv[pl.ds(g * 16, 16)] = (
                    nv[pl.ds(g * 16, 16)] + _dots16(nbuf, vbuf, scr, g)
                )
                return 0

            lax.fori_loop(0, GRPS, ngrp, 0)
            return 0

        lax.fori_loop(0, K, kb, 0)

        pltpu.sync_copy(pv, pdot.at[pl.ds(off, CHUNK)])
        pltpu.sync_copy(nv, ndot.at[pl.ds(off, CHUNK)])


def _tc_body(p_ref, n_ref, o_ref):
    p = p_ref[...]
    n = n_ref[...]
    lp = jnp.minimum(p, 0.0) - jnp.log1p(jnp.exp(-jnp.abs(p)))
    ln = jnp.minimum(-n, 0.0) - jnp.log1p(jnp.exp(-jnp.abs(n)))
    o_ref[0, 0] = -jnp.sum(lp + ln) * (1.0 / B)


_tc_loss = pl.pallas_call(
    _tc_body,
    out_shape=jax.ShapeDtypeStruct((1, 1), jnp.float32),
    out_specs=pl.BlockSpec(memory_space=pltpu.SMEM),
)


def kernel(target, context, negative_word_batch, input_emb, output_emb):
    neg_t = jnp.transpose(negative_word_batch)
    pdot, ndot = _sc_dots(target, context, neg_t, input_emb, output_emb)
    out = _tc_loss(pdot.reshape(128, 128), ndot.reshape(128, 128))
    return out.reshape(())
